# baseline (speedup 1.0000x reference)
import functools

import jax
import jax.numpy as jnp
from jax import lax
from jax.experimental import pallas as pl
from jax.experimental.pallas import tpu as pltpu
from jax.experimental.pallas import tpu_sc as plsc

_NUM_CORES = 2
_NUM_SUBCORES = 16
_NUM_WORKERS = _NUM_CORES * _NUM_SUBCORES

_NCHUNK = 4
_NBUF = 5
_GDEPTH = 3


@functools.lru_cache(maxsize=None)
def _build_sc_embed(BS, NT, D, n_per_w):
    n_c = n_per_w // _NCHUNK
    mesh = plsc.VectorSubcoreMesh(
        core_axis_name="c", subcore_axis_name="s", num_cores=_NUM_CORES,
        num_subcores=_NUM_SUBCORES)

    @functools.partial(
        pl.kernel,
        mesh=mesh,
        compiler_params=pltpu.CompilerParams(
            use_tc_tiling_on_sc=False, needs_layout_passes=False),
        out_type=jax.ShapeDtypeStruct((BS, NT, D), jnp.float32),
        scratch_types=[
            pltpu.VMEM((n_c, NT), jnp.int32),
            pltpu.VMEM((NT, n_per_w), jnp.int32),
            pltpu.VMEM((_NBUF, n_c, D), jnp.float32),
            pltpu.SemaphoreType.DMA,
        ] + [pltpu.SemaphoreType.DMA] * _NBUF
          + [pltpu.SemaphoreType.DMA] * _NBUF,
    )
    def sc_embed(idx_hbm, *rest):
        tabs = rest[:NT]
        out_hbm = rest[NT]
        idx_q = rest[NT + 1]
        idx_t = rest[NT + 2]
        rows_v = rest[NT + 3]
        isem = rest[NT + 4]
        gsems = rest[NT + 5:NT + 5 + _NBUF]
        wsems = rest[NT + 5 + _NBUF:NT + 5 + 2 * _NBUF]
        wid = lax.axis_index("s") * _NUM_CORES + lax.axis_index("c")
        base = wid * n_per_w

        for q in range(_NCHUNK):
            pltpu.async_copy(
                idx_hbm.at[pl.ds(base + q * n_c, n_c)], idx_q, isem).wait()

            def ext_v(v, _, q=q):
                col = jnp.full((16,), v, jnp.int32)

                def ext_m(m5, _, q=q, col=col, v=v):
                    for mm in range(5):
                        m = m5 * 5 + mm
                        rows = lax.iota(jnp.int32, 16) + m * 16
                        vals = plsc.load_gather(idx_q, [rows, col])
                        idx_t[v, pl.ds(q * n_c + m * 16, 16)] = vals
                    return 0

                lax.fori_loop(0, n_c // 80, ext_m, 0)
                return 0

            lax.fori_loop(0, NT, ext_v, 0)

        units = [(q, v) for q in range(_NCHUNK) for v in range(NT)]
        n_u = len(units)
        gcps = [None] * n_u
        wcps = [None] * n_u

        def gather(u):
            q, v = units[u]
            gcps[u] = pltpu.async_copy(
                tabs[v].at[idx_t.at[v, pl.ds(q * n_c, n_c)]],
                rows_v.at[u % _NBUF], gsems[u % _NBUF])

        def write(u):
            q, v = units[u]
            wcps[u] = pltpu.async_copy(
                rows_v.at[u % _NBUF],
                out_hbm.at[pl.ds(base + q * n_c, n_c), v],
                wsems[u % _NBUF])

        for u in range(n_u):
            if u >= _NBUF:
                wcps[u - _NBUF].wait()
            gather(u)
            if u >= _GDEPTH:
                gcps[u - _GDEPTH].wait()
                write(u - _GDEPTH)
        for u in range(n_u - _GDEPTH, n_u):
            gcps[u].wait()
            write(u)
        for u in range(n_u - _NBUF, n_u):
            wcps[u].wait()

    return sc_embed


def kernel(x, tables):
    B, S, NX = x.shape
    D = tables[0].shape[1]
    NT = len(tables)
    BS = B * S
    LAG = tables[NX + 1].shape[0] - 1
    n_per_w = BS // _NUM_WORKERS

    s_row = jnp.arange(S, dtype=jnp.int32)
    pf = jnp.concatenate(
        [jnp.zeros(S - LAG, jnp.int32), jnp.arange(1, LAG + 1, dtype=jnp.int32)])
    isf = jnp.concatenate(
        [jnp.zeros(S - LAG, jnp.int32), jnp.ones(LAG, jnp.int32)])
    pos_pat = jnp.stack([s_row, pf, isf], axis=1)
    pos_cols = jnp.broadcast_to(pos_pat[None], (B, S, 3)).reshape(BS, 3)
    idx2 = jnp.concatenate(
        [x.astype(jnp.int32).reshape(BS, NX), pos_cols], axis=1)

    out = _build_sc_embed(BS, NT, D, n_per_w)(idx2, *tables)
    return out.reshape(B, S, NT, D)


# speedup vs baseline: 1.0014x; 1.0014x over previous
import functools

import jax
import jax.numpy as jnp
from jax import lax
from jax.experimental import pallas as pl
from jax.experimental.pallas import tpu as pltpu
from jax.experimental.pallas import tpu_sc as plsc

_NUM_CORES = 2
_NUM_SUBCORES = 16
_NUM_WORKERS = _NUM_CORES * _NUM_SUBCORES

_NCHUNK = 4
_NBUF = 5
_GDEPTH = 4


@functools.lru_cache(maxsize=None)
def _build_sc_embed(B, S, NX, NT, D, n_per_w):
    BS = B * S
    n_c = n_per_w // _NCHUNK
    b_per_w = n_per_w // S
    b_per_q = b_per_w // _NCHUNK
    mesh = plsc.VectorSubcoreMesh(
        core_axis_name="c", subcore_axis_name="s", num_cores=_NUM_CORES,
        num_subcores=_NUM_SUBCORES)

    @functools.partial(
        pl.kernel,
        mesh=mesh,
        compiler_params=pltpu.CompilerParams(
            use_tc_tiling_on_sc=False, needs_layout_passes=False),
        out_type=jax.ShapeDtypeStruct((BS, NT, D), jnp.float32),
        scratch_types=[
            pltpu.VMEM((b_per_q, S, NX), jnp.int32),
            pltpu.VMEM((S, 4), jnp.int32),
            pltpu.VMEM((NT, n_per_w), jnp.int32),
            pltpu.VMEM((_NBUF, n_c, D), jnp.float32),
            pltpu.SemaphoreType.DMA,
        ] + [pltpu.SemaphoreType.DMA] * _NBUF
          + [pltpu.SemaphoreType.DMA] * _NBUF,
    )
    def sc_embed(x_hbm, pos_hbm, *rest):
        tabs = rest[:NT]
        out_hbm = rest[NT]
        idx_q = rest[NT + 1]
        pos_v = rest[NT + 2]
        idx_t = rest[NT + 3]
        rows_v = rest[NT + 4]
        isem = rest[NT + 5]
        gsems = rest[NT + 6:NT + 6 + _NBUF]
        wsems = rest[NT + 6 + _NBUF:NT + 6 + 2 * _NBUF]
        wid = lax.axis_index("s") * _NUM_CORES + lax.axis_index("c")
        base = wid * n_per_w
        b0 = wid * b_per_w

        pltpu.async_copy(pos_hbm, pos_v, isem).wait()

        for q in range(_NCHUNK):
            pltpu.async_copy(
                x_hbm.at[pl.ds(b0 + q * b_per_q, b_per_q)], idx_q, isem
            ).wait()

            def ext_v(v, _, q=q):
                col = jnp.full((16,), v, jnp.int32)

                def ext_m(m5, _, q=q, col=col, v=v):
                    s_splat = jnp.full((16,), S, jnp.int32)
                    for mm in range(5):
                        m = m5 * 5 + mm
                        p = lax.iota(jnp.int32, 16) + m * 16
                        vals = plsc.load_gather(
                            idx_q, [lax.div(p, s_splat),
                                    lax.rem(p, s_splat), col])
                        idx_t[v, pl.ds(q * n_c + m * 16, 16)] = vals
                    return 0

                lax.fori_loop(0, n_c // 80, ext_m, 0)
                return 0

            lax.fori_loop(0, NX, ext_v, 0)

        for k in range(NT - NX):
            colk = jnp.full((16,), k, jnp.int32)

            def ext_p(m, _, k=k, colk=colk):
                r = lax.rem(lax.iota(jnp.int32, 16) + m * 16,
                            jnp.full((16,), S, jnp.int32))
                vals = plsc.load_gather(pos_v, [r, colk])
                idx_t[NX + k, pl.ds(m * 16, 16)] = vals
                return 0

            lax.fori_loop(0, n_per_w // 16, ext_p, 0)

        units = [(q, v) for q in range(_NCHUNK) for v in range(NT)]
        n_u = len(units)
        gcps = [None] * n_u
        wcps = [None] * n_u

        def gather(u):
            q, v = units[u]
            gcps[u] = pltpu.async_copy(
                tabs[v].at[idx_t.at[v, pl.ds(q * n_c, n_c)]],
                rows_v.at[u % _NBUF], gsems[u % _NBUF])

        def write(u):
            q, v = units[u]
            wcps[u] = pltpu.async_copy(
                rows_v.at[u % _NBUF],
                out_hbm.at[pl.ds(base + q * n_c, n_c), v],
                wsems[u % _NBUF])

        for u in range(n_u):
            if u >= _NBUF:
                wcps[u - _NBUF].wait()
            gather(u)
            if u >= _GDEPTH:
                gcps[u - _GDEPTH].wait()
                write(u - _GDEPTH)
        for u in range(n_u - _GDEPTH, n_u):
            gcps[u].wait()
            write(u)
        for u in range(n_u - _NBUF, n_u):
            wcps[u].wait()

    return sc_embed


def kernel(x, tables):
    B, S, NX = x.shape
    D = tables[0].shape[1]
    NT = len(tables)
    BS = B * S
    LAG = tables[NX + 1].shape[0] - 1
    n_per_w = BS // _NUM_WORKERS

    s_row = jnp.arange(S, dtype=jnp.int32)
    pf = jnp.concatenate(
        [jnp.zeros(S - LAG, jnp.int32), jnp.arange(1, LAG + 1, dtype=jnp.int32)])
    isf = jnp.concatenate(
        [jnp.zeros(S - LAG, jnp.int32), jnp.ones(LAG, jnp.int32)])
    pos_pat = jnp.stack([s_row, pf, isf, jnp.zeros(S, jnp.int32)], axis=1)

    out = _build_sc_embed(B, S, NX, NT, D, n_per_w)(
        x.astype(jnp.int32), pos_pat, *tables)
    return out.reshape(B, S, NT, D)


# speedup vs baseline: 1.0025x; 1.0011x over previous
import functools

import jax
import jax.numpy as jnp
from jax import lax
from jax.experimental import pallas as pl
from jax.experimental.pallas import tpu as pltpu
from jax.experimental.pallas import tpu_sc as plsc

_NUM_CORES = 2
_NUM_SUBCORES = 16
_NUM_WORKERS = _NUM_CORES * _NUM_SUBCORES

_NCHUNK = 4
_NBUF = 5
_GDEPTH = 4


@functools.lru_cache(maxsize=None)
def _build_sc_embed(B, S, NX, NT, D, n_per_w):
    BS = B * S
    n_c = n_per_w // _NCHUNK
    b_per_w = n_per_w // S
    b_per_q = b_per_w // _NCHUNK
    mesh = plsc.VectorSubcoreMesh(
        core_axis_name="c", subcore_axis_name="s", num_cores=_NUM_CORES,
        num_subcores=_NUM_SUBCORES)

    @functools.partial(
        pl.kernel,
        mesh=mesh,
        compiler_params=pltpu.CompilerParams(
            use_tc_tiling_on_sc=False, needs_layout_passes=False),
        out_type=jax.ShapeDtypeStruct((BS, NT, D), jnp.float32),
        scratch_types=[
            pltpu.VMEM((b_per_q, S, NX), jnp.int32),
            pltpu.VMEM((S, 4), jnp.int32),
            pltpu.VMEM((NT, n_per_w), jnp.int32),
            pltpu.VMEM((_NBUF, n_c, D), jnp.float32),
            pltpu.SemaphoreType.DMA,
        ] + [pltpu.SemaphoreType.DMA] * _NBUF
          + [pltpu.SemaphoreType.DMA] * _NBUF,
    )
    def sc_embed(x_hbm, pos_hbm, *rest):
        tabs = rest[:NT]
        out_hbm = rest[NT]
        idx_q = rest[NT + 1]
        pos_v = rest[NT + 2]
        idx_t = rest[NT + 3]
        rows_v = rest[NT + 4]
        isem = rest[NT + 5]
        gsems = rest[NT + 6:NT + 6 + _NBUF]
        wsems = rest[NT + 6 + _NBUF:NT + 6 + 2 * _NBUF]
        wid = lax.axis_index("s") * _NUM_CORES + lax.axis_index("c")
        base = wid * n_per_w
        b0 = wid * b_per_w

        pltpu.async_copy(pos_hbm, pos_v, isem).wait()

        for q in range(_NCHUNK):
            pltpu.async_copy(
                x_hbm.at[pl.ds(b0 + q * b_per_q, b_per_q)], idx_q, isem
            ).wait()

            def ext_v(v, _, q=q):
                col = jnp.full((16,), v, jnp.int32)

                def ext_m(m5, _, q=q, col=col, v=v):
                    s_splat = jnp.full((16,), S, jnp.int32)
                    for mm in range(5):
                        m = m5 * 5 + mm
                        p = lax.iota(jnp.int32, 16) + m * 16
                        vals = plsc.load_gather(
                            idx_q, [lax.div(p, s_splat),
                                    lax.rem(p, s_splat), col])
                        idx_t[v, pl.ds(q * n_c + m * 16, 16)] = vals
                    return 0

                lax.fori_loop(0, n_c // 80, ext_m, 0)
                return 0

            lax.fori_loop(0, NX, ext_v, 0)

        for k in range(NT - NX):
            colk = jnp.full((16,), k, jnp.int32)

            def ext_p(m, _, k=k, colk=colk):
                r = lax.rem(lax.iota(jnp.int32, 16) + m * 16,
                            jnp.full((16,), S, jnp.int32))
                vals = plsc.load_gather(pos_v, [r, colk])
                idx_t[NX + k, pl.ds(m * 16, 16)] = vals
                return 0

            lax.fori_loop(0, n_per_w // 16, ext_p, 0)

        units = [(q, v) for q in range(_NCHUNK) for v in range(NT)]
        n_u = len(units)
        gcps = [None] * n_u
        wcps = [None] * n_u

        def gather(u):
            q, v = units[u]
            gcps[u] = pltpu.async_copy(
                tabs[v].at[idx_t.at[v, pl.ds(q * n_c, n_c)]],
                rows_v.at[u % _NBUF], gsems[u % _NBUF])

        def write(u):
            q, v = units[u]
            wcps[u] = pltpu.async_copy(
                rows_v.at[u % _NBUF],
                out_hbm.at[pl.ds(base + q * n_c, n_c), v],
                wsems[u % _NBUF])

        for u in range(n_u):
            if u >= _NBUF:
                wcps[u - _NBUF].wait()
            gather(u)
            if u >= _GDEPTH:
                gcps[u - _GDEPTH].wait()
                write(u - _GDEPTH)
        for u in range(n_u - _GDEPTH, n_u):
            gcps[u].wait()
            write(u)
        for u in range(n_u - _NBUF, n_u):
            wcps[u].wait()

    return sc_embed


def kernel(x, tables):
    B, S, NX = x.shape
    D = tables[0].shape[1]
    NT = len(tables)
    BS = B * S
    LAG = tables[NX + 1].shape[0] - 1
    n_per_w = BS // _NUM_WORKERS

    s_row = jnp.arange(S, dtype=jnp.int32)
    pf = jnp.concatenate(
        [jnp.zeros(S - LAG, jnp.int32), jnp.arange(1, LAG + 1, dtype=jnp.int32)])
    isf = jnp.concatenate(
        [jnp.zeros(S - LAG, jnp.int32), jnp.ones(LAG, jnp.int32)])
    pos_pat = jnp.stack([s_row, pf, isf, jnp.zeros(S, jnp.int32)], axis=1)

    tabs_1d = jax.lax.optimization_barrier(
        tuple(jnp.reshape(t, (-1,)) for t in tables))
    tabs = [jnp.reshape(t1, t.shape) for t1, t in zip(tabs_1d, tables)]

    out = _build_sc_embed(B, S, NX, NT, D, n_per_w)(
        x.astype(jnp.int32), pos_pat, *tabs)
    return out.reshape(B, S, NT, D)


# speedup vs baseline: 1.3508x; 1.3475x over previous
import functools

import jax
import jax.numpy as jnp
from jax import lax
from jax.experimental import pallas as pl
from jax.experimental.pallas import tpu as pltpu
from jax.experimental.pallas import tpu_sc as plsc

_NUM_CORES = 2
_NUM_SUBCORES = 16
_NUM_WORKERS = _NUM_CORES * _NUM_SUBCORES

_NSTAGE = 4
_NCHUNK = 5
_NBUF = 6
_GDEPTH = 5
_SPAD = 64


@functools.lru_cache(maxsize=None)
def _build_sc_embed(B, S, NX, NT, D, n_per_w):
    BS = B * S
    NP = NT - NX
    n_c = n_per_w // _NCHUNK
    n_s = n_per_w // _NSTAGE
    b_per_w = n_per_w // S
    b_per_q = b_per_w // _NSTAGE
    reps = n_per_w // S
    mesh = plsc.VectorSubcoreMesh(
        core_axis_name="c", subcore_axis_name="s", num_cores=_NUM_CORES,
        num_subcores=_NUM_SUBCORES)

    @functools.partial(
        pl.kernel,
        mesh=mesh,
        compiler_params=pltpu.CompilerParams(
            use_tc_tiling_on_sc=False, needs_layout_passes=False),
        out_type=jax.ShapeDtypeStruct((BS, NT, D), jnp.float32),
        scratch_types=[
            pltpu.VMEM((b_per_q, S, NX), jnp.int32),
            pltpu.VMEM((_SPAD, 4), jnp.int32),
            pltpu.VMEM((NP, _SPAD), jnp.int32),
            pltpu.VMEM((NP, _SPAD, D), jnp.float32),
            pltpu.VMEM((NX, n_per_w), jnp.int32),
            pltpu.VMEM((_NBUF, n_c, D), jnp.float32),
            pltpu.SemaphoreType.DMA,
            pltpu.SemaphoreType.DMA,
            pltpu.SemaphoreType.DMA,
        ] + [pltpu.SemaphoreType.DMA] * _NBUF
          + [pltpu.SemaphoreType.DMA] * _NBUF,
    )
    def sc_embed(x_hbm, pos_hbm, *rest):
        tabs = rest[:NT]
        out_hbm = rest[NT]
        idx_q, pos_v, pidx, pos_rows = rest[NT + 1:NT + 5]
        idx_t = rest[NT + 5]
        rows_v = rest[NT + 6]
        isem, psem, p2sem = rest[NT + 7:NT + 10]
        gsems = rest[NT + 10:NT + 10 + _NBUF]
        wsems = rest[NT + 10 + _NBUF:NT + 10 + 2 * _NBUF]
        wid = lax.axis_index("s") * _NUM_CORES + lax.axis_index("c")
        base = wid * n_per_w
        b0 = wid * b_per_w

        pltpu.async_copy(pos_hbm, pos_v, isem).wait()
        for k in range(NP):
            colk = jnp.full((16,), k, jnp.int32)
            for j in range(_SPAD // 16):
                rows = lax.iota(jnp.int32, 16) + j * 16
                pidx[k, pl.ds(j * 16, 16)] = plsc.load_gather(
                    pos_v, [rows, colk])
        pcps = [
            pltpu.async_copy(
                tabs[NX + k].at[pidx.at[k]], pos_rows.at[k], psem)
            for k in range(NP)
        ]

        for q in range(_NSTAGE):
            pltpu.async_copy(
                x_hbm.at[pl.ds(b0 + q * b_per_q, b_per_q)], idx_q, isem
            ).wait()

            def ext_v(v, _, q=q):
                col = jnp.full((16,), v, jnp.int32)

                def ext_m(m5, _, q=q, col=col, v=v):
                    s_splat = jnp.full((16,), S, jnp.int32)
                    for mm in range(5):
                        m = m5 * 5 + mm
                        p = lax.iota(jnp.int32, 16) + m * 16
                        vals = plsc.load_gather(
                            idx_q, [lax.div(p, s_splat),
                                    lax.rem(p, s_splat), col])
                        idx_t[v, pl.ds(q * n_s + m * 16, 16)] = vals
                    return 0

                lax.fori_loop(0, n_s // 80, ext_m, 0)
                return 0

            lax.fori_loop(0, NX, ext_v, 0)

        for cp in pcps:
            cp.wait()
        pwcps = []
        for k in range(NP):
            for r in range(reps):
                pwcps.append(pltpu.async_copy(
                    pos_rows.at[k, pl.ds(0, S)],
                    out_hbm.at[pl.ds(base + r * S, S), NX + k], p2sem))

        units = [(q, v) for q in range(_NCHUNK) for v in range(NX)]
        n_u = len(units)
        gcps = [None] * n_u
        wcps = [None] * n_u

        def gather(u):
            q, v = units[u]
            gcps[u] = pltpu.async_copy(
                tabs[v].at[idx_t.at[v, pl.ds(q * n_c, n_c)]],
                rows_v.at[u % _NBUF], gsems[u % _NBUF])

        def write(u):
            q, v = units[u]
            wcps[u] = pltpu.async_copy(
                rows_v.at[u % _NBUF],
                out_hbm.at[pl.ds(base + q * n_c, n_c), v],
                wsems[u % _NBUF])

        for u in range(n_u):
            if u >= _NBUF:
                wcps[u - _NBUF].wait()
            gather(u)
            if u >= _GDEPTH:
                gcps[u - _GDEPTH].wait()
                write(u - _GDEPTH)
        for u in range(n_u - _GDEPTH, n_u):
            gcps[u].wait()
            write(u)
        for u in range(n_u - _NBUF, n_u):
            wcps[u].wait()
        for cp in pwcps:
            cp.wait()

    return sc_embed


def kernel(x, tables):
    B, S, NX = x.shape
    D = tables[0].shape[1]
    NT = len(tables)
    BS = B * S
    LAG = tables[NX + 1].shape[0] - 1
    n_per_w = BS // _NUM_WORKERS

    s_row = jnp.arange(S, dtype=jnp.int32)
    pf = jnp.concatenate(
        [jnp.zeros(S - LAG, jnp.int32), jnp.arange(1, LAG + 1, dtype=jnp.int32)])
    isf = jnp.concatenate(
        [jnp.zeros(S - LAG, jnp.int32), jnp.ones(LAG, jnp.int32)])
    pos_pat = jnp.zeros((_SPAD, 4), jnp.int32)
    pos_pat = pos_pat.at[:S, 0].set(s_row)
    pos_pat = pos_pat.at[:S, 1].set(pf)
    pos_pat = pos_pat.at[:S, 2].set(isf)

    out = _build_sc_embed(B, S, NX, NT, D, n_per_w)(
        x.astype(jnp.int32), pos_pat, *tables)
    return out.reshape(B, S, NT, D)
